# Initial kernel scaffold; baseline (speedup 1.0000x reference)
import jax
import jax.numpy as jnp
from jax.experimental import pallas as pl


def kernel(time, table):
    raise NotImplementedError("write your pallas kernel here")


# speedup vs baseline: 6.0767x; 6.0767x over previous
import jax
import jax.numpy as jnp
from jax.experimental import pallas as pl

_MAX_REL_POS = 128
_DIM = 64


def _bcast_body(tbl_ref, out_ref):
    i = pl.program_id(0)
    row = tbl_ref[pl.ds(i, 1), :]
    out_ref[...] = jnp.broadcast_to(row[:, None, :], out_ref.shape)


def kernel(time, table):
    _, seq_len = time.shape
    rows = 2 * _MAX_REL_POS + 1
    out = pl.pallas_call(
        _bcast_body,
        grid=(rows,),
        in_specs=[pl.BlockSpec((rows, _DIM), lambda i: (0, 0))],
        out_specs=pl.BlockSpec((1, seq_len, _DIM), lambda i: (i, 0, 0)),
        out_shape=jax.ShapeDtypeStruct((rows, seq_len, _DIM), jnp.float32),
    )(table)
    return out
